# Initial kernel scaffold; baseline (speedup 1.0000x reference)
import jax
import jax.numpy as jnp
from jax.experimental import pallas as pl


def kernel(x, edge_index, w1l, b1, w1r, w2l, b2, w2r, w3l, b3, w3r, wc, bc):
    raise NotImplementedError("write your pallas kernel here")


# speedup vs baseline: 54.8990x; 54.8990x over previous
import functools

import jax
import jax.numpy as jnp
from jax import lax
from jax.experimental import pallas as pl
from jax.experimental.pallas import tpu as pltpu
from jax.experimental.pallas import tpu_sc as plsc

_LANES = 128
_KB = 16
_NC = 2
_NS = 16
_D = 8


def _sc_aggregate(table, src_rows, dst_rows, zeros, n_acc):
  r_pad = src_rows.shape[0]
  rows_per_tile = r_pad // (_NC * _NS)
  zr = n_acc // _NS
  mesh = plsc.VectorSubcoreMesh(core_axis_name="c", subcore_axis_name="s")

  @functools.partial(
      pl.kernel, mesh=mesh,
      compiler_params=pltpu.CompilerParams(use_tc_tiling_on_sc=False),
      out_type=jax.ShapeDtypeStruct((_NC, n_acc, _D), jnp.float32),
      scratch_types=[
          pltpu.VMEM((_KB, _LANES), jnp.int32),
          pltpu.VMEM((_KB, _LANES), jnp.int32),
          pltpu.VMEM((_KB, _LANES, _D), jnp.float32),
          pltpu.VMEM_SHARED((n_acc, _D), jnp.float32),
          pltpu.SemaphoreType.DMA,
      ],
  )
  def agg(table_hbm, src_hbm, dst_hbm, z_hbm, out_hbm, srcv, dstv, rowsv,
          acc, sem):
    cid = lax.axis_index("c")
    sid = lax.axis_index("s")
    wid = sid * _NC + cid
    tile_slice = pl.ds(sid * zr, zr)
    pltpu.sync_copy(z_hbm.at[tile_slice], acc.at[tile_slice])
    plsc.subcore_barrier()
    base = wid * rows_per_tile

    @pl.loop(0, rows_per_tile, step=_KB)
    def _(g):
      pltpu.sync_copy(src_hbm.at[pl.ds(base + g, _KB)], srcv)
      pltpu.sync_copy(dst_hbm.at[pl.ds(base + g, _KB)], dstv)
      cps = [pltpu.async_copy(table_hbm.at[srcv.at[j]], rowsv.at[j], sem)
             for j in range(_KB)]
      for j in range(_KB):
        cps[j].wait()
        pltpu.sync_copy(rowsv.at[j], acc.at[dstv.at[j]], add=True)

    plsc.subcore_barrier()
    pltpu.sync_copy(acc.at[tile_slice], out_hbm.at[cid, tile_slice])

  return agg(table, src_rows, dst_rows, zeros)


def _matcols(h, w_ref, din):
  acc = h[:, 0:1] * w_ref[0:1, :]
  for k in range(1, din):
    acc = acc + h[:, k:k + 1] * w_ref[k:k + 1, :]
  return acc


_BN = 2048


def _combine1(p0, p1, x, w1l, b1, w1r):
  n = x.shape[0]

  def body(p0r, p1r, xr, wl, bl, wr, h_ref, dinv_ref):
    s = p0r[...] + p1r[...]
    dinv = 1.0 / jnp.maximum(s[:, 1:2], 1.0)
    mean = s[:, 0:1] * dinv
    z = mean * wl[0:1, :] + bl[0:1, :] + xr[...] * wr[0:1, :]
    h = jnp.maximum(z, 0.0)
    h_ref[...] = jnp.concatenate([h, jnp.zeros((h.shape[0], _D - 4), h.dtype)],
                                 axis=1)
    dinv_ref[...] = dinv

  return pl.pallas_call(
      body,
      grid=(pl.cdiv(n, _BN),),
      in_specs=[
          pl.BlockSpec((_BN, _D), lambda i: (i, 0)),
          pl.BlockSpec((_BN, _D), lambda i: (i, 0)),
          pl.BlockSpec((_BN, 1), lambda i: (i, 0)),
          pl.BlockSpec((1, 4), lambda i: (0, 0)),
          pl.BlockSpec((1, 4), lambda i: (0, 0)),
          pl.BlockSpec((1, 4), lambda i: (0, 0)),
      ],
      out_specs=[pl.BlockSpec((_BN, _D), lambda i: (i, 0)),
                 pl.BlockSpec((_BN, 1), lambda i: (i, 0))],
      out_shape=[jax.ShapeDtypeStruct((n, _D), jnp.float32),
                 jax.ShapeDtypeStruct((n, 1), jnp.float32)],
  )(p0, p1, x, w1l, b1, w1r)


def _combine_mid(p0, p1, dinv, h_prev, wl, b, wr):
  n = h_prev.shape[0]
  din, dout = wl.shape

  def body(p0r, p1r, dvr, hr, wlr, br, wrr, h_ref):
    mean = (p0r[:, :din] + p1r[:, :din]) * dvr[...]
    z = br[0:1, :] + _matcols(mean, wlr, din) + _matcols(hr[:, :din], wrr, din)
    h = jnp.maximum(z, 0.0)
    h_ref[...] = jnp.concatenate(
        [h, jnp.zeros((h.shape[0], _D - dout), h.dtype)], axis=1)

  return pl.pallas_call(
      body,
      grid=(pl.cdiv(n, _BN),),
      in_specs=[
          pl.BlockSpec((_BN, _D), lambda i: (i, 0)),
          pl.BlockSpec((_BN, _D), lambda i: (i, 0)),
          pl.BlockSpec((_BN, 1), lambda i: (i, 0)),
          pl.BlockSpec((_BN, _D), lambda i: (i, 0)),
          pl.BlockSpec((din, dout), lambda i: (0, 0)),
          pl.BlockSpec((1, dout), lambda i: (0, 0)),
          pl.BlockSpec((din, dout), lambda i: (0, 0)),
      ],
      out_specs=[pl.BlockSpec((_BN, _D), lambda i: (i, 0))],
      out_shape=[jax.ShapeDtypeStruct((n, _D), jnp.float32)],
  )(p0, p1, dinv, h_prev, wl, b, wr)[0]


def _combine_last(p0, p1, dinv, h_prev, wl, b, wr, wc, bc):
  n = h_prev.shape[0]
  din, dout = wl.shape
  dcls = wc.shape[1]

  def body(p0r, p1r, dvr, hr, wlr, br, wrr, wcr, bcr, out_ref, h_ref):
    mean = (p0r[:, :din] + p1r[:, :din]) * dvr[...]
    z = br[0:1, :] + _matcols(mean, wlr, din) + _matcols(hr[:, :din], wrr, din)
    h = jnp.maximum(z, 0.0)
    h_ref[...] = h
    out_ref[...] = bcr[0:1, :] + _matcols(h, wcr, dout)

  return pl.pallas_call(
      body,
      grid=(pl.cdiv(n, _BN),),
      in_specs=[
          pl.BlockSpec((_BN, _D), lambda i: (i, 0)),
          pl.BlockSpec((_BN, _D), lambda i: (i, 0)),
          pl.BlockSpec((_BN, 1), lambda i: (i, 0)),
          pl.BlockSpec((_BN, _D), lambda i: (i, 0)),
          pl.BlockSpec((din, dout), lambda i: (0, 0)),
          pl.BlockSpec((1, dout), lambda i: (0, 0)),
          pl.BlockSpec((din, dout), lambda i: (0, 0)),
          pl.BlockSpec((dout, dcls), lambda i: (0, 0)),
          pl.BlockSpec((1, dcls), lambda i: (0, 0)),
      ],
      out_specs=[pl.BlockSpec((_BN, dcls), lambda i: (i, 0)),
                 pl.BlockSpec((_BN, dout), lambda i: (i, 0))],
      out_shape=[jax.ShapeDtypeStruct((n, dcls), jnp.float32),
                 jax.ShapeDtypeStruct((n, dout), jnp.float32)],
  )(p0, p1, dinv, h_prev, wl, b, wr, wc, bc)


def kernel(x, edge_index, w1l, b1, w1r, w2l, b2, w2r, w3l, b3, w3r, wc, bc):
  n = x.shape[0]
  e = edge_index.shape[1]

  group = _NC * _NS * _KB * _LANES
  e_pad = ((e + group - 1) // group) * group
  n_acc = ((n + _LANES) // _LANES) * _LANES

  src = edge_index[0]
  dst = edge_index[1]
  npad = e_pad - e
  if npad:
    src = jnp.concatenate([src, jnp.zeros((npad,), jnp.int32)])
    pad_dst = n + (jnp.arange(npad, dtype=jnp.int32) % (n_acc - n))
    dst = jnp.concatenate([dst, pad_dst])
  src_rows = src.reshape(-1, _LANES)
  dst_rows = dst.reshape(-1, _LANES)

  xa = jnp.concatenate(
      [x, jnp.ones((n, 1), jnp.float32), jnp.zeros((n, _D - 2), jnp.float32)],
      axis=1)
  z8 = jnp.zeros((n_acc, _D), jnp.float32)

  b1r, b2r, b3r, bcr = (v.reshape(1, -1) for v in (b1, b2, b3, bc))

  p = _sc_aggregate(xa, src_rows, dst_rows, z8, n_acc)
  h1, dinv = _combine1(p[0, :n], p[1, :n], x, w1l, b1r, w1r)
  p = _sc_aggregate(h1, src_rows, dst_rows, z8, n_acc)
  h2 = _combine_mid(p[0, :n], p[1, :n], dinv, h1, w2l, b2r, w2r)
  p = _sc_aggregate(h2, src_rows, dst_rows, z8, n_acc)
  out, h3 = _combine_last(p[0, :n], p[1, :n], dinv, h2, w3l, b3r, w3r, wc, bcr)
  return (out, h3)


# speedup vs baseline: 70.3314x; 1.2811x over previous
import functools

import jax
import jax.numpy as jnp
from jax import lax
from jax.experimental import pallas as pl
from jax.experimental.pallas import tpu as pltpu
from jax.experimental.pallas import tpu_sc as plsc

_LANES = 128
_KB = 16
_NC = 2
_NS = 16
_D = 8
_PK = _LANES // _D


def _sc_aggregate(table, src_rows, dst_rows, zeros, n_acc):
  r_pad = src_rows.shape[0]
  rows_per_tile = r_pad // (_NC * _NS)
  zr = n_acc // _NS
  mesh = plsc.VectorSubcoreMesh(core_axis_name="c", subcore_axis_name="s")

  @functools.partial(
      pl.kernel, mesh=mesh,
      compiler_params=pltpu.CompilerParams(use_tc_tiling_on_sc=False),
      out_type=jax.ShapeDtypeStruct((_NC, n_acc, _D), jnp.float32),
      scratch_types=[
          pltpu.VMEM((_KB, _LANES), jnp.int32),
          pltpu.VMEM((_KB, _LANES), jnp.int32),
          pltpu.VMEM((_KB, _LANES, _D), jnp.float32),
          pltpu.VMEM_SHARED((n_acc, _D), jnp.float32),
          pltpu.SemaphoreType.DMA,
      ],
  )
  def agg(table_hbm, src_hbm, dst_hbm, z_hbm, out_hbm, srcv, dstv, rowsv,
          acc, sem):
    cid = lax.axis_index("c")
    sid = lax.axis_index("s")
    wid = sid * _NC + cid
    tile_slice = pl.ds(sid * zr, zr)
    pltpu.sync_copy(z_hbm.at[tile_slice], acc.at[tile_slice])
    plsc.subcore_barrier()
    base = wid * rows_per_tile

    @pl.loop(0, rows_per_tile, step=_KB)
    def _(g):
      pltpu.sync_copy(src_hbm.at[pl.ds(base + g, _KB)], srcv)
      pltpu.sync_copy(dst_hbm.at[pl.ds(base + g, _KB)], dstv)
      cps = [pltpu.async_copy(table_hbm.at[srcv.at[j]], rowsv.at[j], sem)
             for j in range(_KB)]
      for j in range(_KB):
        cps[j].wait()
        pltpu.sync_copy(rowsv.at[j], acc.at[dstv.at[j]], add=True)

    plsc.subcore_barrier()
    pltpu.sync_copy(acc.at[tile_slice], out_hbm.at[cid, tile_slice])

  return agg(table, src_rows, dst_rows, zeros)


def _blockdiag(w):
  din, dout = w.shape
  w8 = jnp.zeros((_D, _D), jnp.float32).at[:din, :dout].set(w)
  return jnp.kron(jnp.eye(_PK, dtype=jnp.float32), w8)


def _brow(b):
  b8 = jnp.zeros((_D,), jnp.float32).at[:b.shape[0]].set(b)
  return jnp.tile(b8, _PK).reshape(1, _LANES)


def _dot(a, b):
  return jax.lax.dot_general(a, b, (((1,), (0,)), ((), ())),
                             precision=jax.lax.Precision.HIGHEST)


def _combine1(p, xa_p, wl_blk, wr_blk, b_row, sel_deg):
  r = xa_p.shape[0]

  def body(pr, xar, wlr, wrr, br, selr, h_ref, dinv_ref):
    s = pr[0] + pr[1]
    dinv = 1.0 / jnp.maximum(_dot(s, selr[...]), 1.0)
    z = _dot(s * dinv, wlr[...]) + br[...] + _dot(xar[...], wrr[...])
    h_ref[...] = jnp.maximum(z, 0.0)
    dinv_ref[...] = dinv

  return pl.pallas_call(
      body,
      out_shape=[jax.ShapeDtypeStruct((r, _LANES), jnp.float32),
                 jax.ShapeDtypeStruct((r, _LANES), jnp.float32)],
  )(p, xa_p, wl_blk, wr_blk, b_row, sel_deg)


def _combine_mid(p, dinv_p, h_prev_p, wl_blk, wr_blk, b_row):
  r = h_prev_p.shape[0]

  def body(pr, dvr, hr, wlr, wrr, br, h_ref):
    mean = (pr[0] + pr[1]) * dvr[...]
    z = _dot(mean, wlr[...]) + br[...] + _dot(hr[...], wrr[...])
    h_ref[...] = jnp.maximum(z, 0.0)

  return pl.pallas_call(
      body,
      out_shape=jax.ShapeDtypeStruct((r, _LANES), jnp.float32),
  )(p, dinv_p, h_prev_p, wl_blk, wr_blk, b_row)


def _combine_last(p, dinv_p, h_prev_p, wl_blk, wr_blk, b_row, wc_blk, bc_row):
  r = h_prev_p.shape[0]

  def body(pr, dvr, hr, wlr, wrr, br, wcr, bcr, out_ref, h_ref):
    mean = (pr[0] + pr[1]) * dvr[...]
    z = _dot(mean, wlr[...]) + br[...] + _dot(hr[...], wrr[...])
    h = jnp.maximum(z, 0.0)
    h_ref[...] = h
    out_ref[...] = _dot(h, wcr[...]) + bcr[...]

  return pl.pallas_call(
      body,
      out_shape=[jax.ShapeDtypeStruct((r, _LANES), jnp.float32),
                 jax.ShapeDtypeStruct((r, _LANES), jnp.float32)],
  )(p, dinv_p, h_prev_p, wl_blk, wr_blk, b_row, wc_blk, bc_row)


def kernel(x, edge_index, w1l, b1, w1r, w2l, b2, w2r, w3l, b3, w3r, wc, bc):
  n = x.shape[0]
  e = edge_index.shape[1]

  group = _NC * _NS * _KB * _LANES
  e_pad = ((e + group - 1) // group) * group
  n_acc = ((n + _LANES) // _LANES) * _LANES
  r = n_acc // _PK

  src = edge_index[0]
  dst = edge_index[1]
  npad = e_pad - e
  if npad:
    src = jnp.concatenate([src, jnp.zeros((npad,), jnp.int32)])
    pad_dst = n + (jnp.arange(npad, dtype=jnp.int32) % (n_acc - n))
    dst = jnp.concatenate([dst, pad_dst])
  src_rows = src.reshape(-1, _LANES)
  dst_rows = dst.reshape(-1, _LANES)

  xa = jnp.concatenate(
      [x, jnp.ones((n, 1), jnp.float32), jnp.zeros((n, _D - 2), jnp.float32)],
      axis=1)
  xa = jnp.concatenate([xa, jnp.zeros((n_acc - n, _D), jnp.float32)], axis=0)
  z8 = jnp.zeros((n_acc, _D), jnp.float32)

  sel8 = jnp.zeros((_D, _D), jnp.float32).at[1, :].set(1.0)
  sel_deg = jnp.kron(jnp.eye(_PK, dtype=jnp.float32), sel8)
  wl1, wr1, wl2, wr2, wl3, wr3, wcb = (
      _blockdiag(w) for w in (w1l, w1r, w2l, w2r, w3l, w3r, wc))
  br1, br2, br3, bcr = (_brow(b) for b in (b1, b2, b3, bc))

  xp = xa.reshape(r, _LANES)
  p = _sc_aggregate(xa, src_rows, dst_rows, z8, n_acc)
  h1p, dinvp = _combine1(p.reshape(_NC, r, _LANES), xp, wl1, wr1, br1, sel_deg)
  p = _sc_aggregate(h1p.reshape(n_acc, _D), src_rows, dst_rows, z8, n_acc)
  h2p = _combine_mid(p.reshape(_NC, r, _LANES), dinvp, h1p, wl2, wr2, br2)
  p = _sc_aggregate(h2p.reshape(n_acc, _D), src_rows, dst_rows, z8, n_acc)
  outp, h3p = _combine_last(p.reshape(_NC, r, _LANES), dinvp, h2p,
                            wl3, wr3, br3, wcb, bcr)
  out = outp.reshape(n_acc, _D)[:n, :2]
  h3 = h3p.reshape(n_acc, _D)[:n, :2]
  return (out, h3)


# speedup vs baseline: 93.5016x; 1.3294x over previous
import functools

import jax
import jax.numpy as jnp
from jax import lax
from jax.experimental import pallas as pl
from jax.experimental.pallas import tpu as pltpu
from jax.experimental.pallas import tpu_sc as plsc

_LANES = 128
_KB = 16
_NC = 2
_NS = 16
_D = 8
_PK = _LANES // _D


def _sc_aggregate(table, src_rows, dst_rows, zeros, n_acc):
  r_pad = src_rows.shape[0]
  rows_per_tile = r_pad // (_NC * _NS)
  zr = n_acc // _NS
  mesh = plsc.VectorSubcoreMesh(core_axis_name="c", subcore_axis_name="s")

  n_chunks = rows_per_tile // _KB

  @functools.partial(
      pl.kernel, mesh=mesh,
      compiler_params=pltpu.CompilerParams(use_tc_tiling_on_sc=False),
      out_type=jax.ShapeDtypeStruct((_NC, n_acc, _D), jnp.float32),
      scratch_types=[
          pltpu.VMEM((2, _KB, _LANES), jnp.int32),
          pltpu.VMEM((2, _KB, _LANES), jnp.int32),
          pltpu.VMEM((2, _KB, _LANES, _D), jnp.float32),
          pltpu.VMEM_SHARED((n_acc, _D), jnp.float32),
          pltpu.SemaphoreType.DMA,
          pltpu.SemaphoreType.DMA,
      ],
  )
  def agg(table_hbm, src_hbm, dst_hbm, z_hbm, out_hbm, srcv, dstv, rowsv,
          acc, sem_g, sem_i):
    cid = lax.axis_index("c")
    sid = lax.axis_index("s")
    wid = sid * _NC + cid
    tile_slice = pl.ds(sid * zr, zr)
    pltpu.sync_copy(z_hbm.at[tile_slice], acc.at[tile_slice])
    plsc.subcore_barrier()
    base = wid * rows_per_tile
    last = n_chunks - 1

    def off(q):
      return base + jnp.minimum(q, last) * _KB

    def issue_idx(q, b):
      pltpu.async_copy(src_hbm.at[pl.ds(off(q), _KB)], srcv.at[b], sem_i)
      pltpu.async_copy(dst_hbm.at[pl.ds(off(q), _KB)], dstv.at[b], sem_i)

    def wait_idx(b):
      pltpu.make_async_copy(src_hbm.at[pl.ds(base, _KB)], srcv.at[b],
                            sem_i).wait()
      pltpu.make_async_copy(dst_hbm.at[pl.ds(base, _KB)], dstv.at[b],
                            sem_i).wait()

    def fire_gathers(b):
      for j in range(_KB):
        pltpu.async_copy(table_hbm.at[srcv.at[b, j]], rowsv.at[b, j], sem_g)

    def wait_gathers(b):
      for j in range(_KB):
        pltpu.make_async_copy(table_hbm.at[srcv.at[b, j]], rowsv.at[b, j],
                              sem_g).wait()

    def scatters(b):
      for j in range(_KB):
        pltpu.sync_copy(rowsv.at[b, j], acc.at[dstv.at[b, j]], add=True)

    pltpu.sync_copy(src_hbm.at[pl.ds(base, _KB)], srcv.at[0])
    pltpu.sync_copy(dst_hbm.at[pl.ds(base, _KB)], dstv.at[0])
    fire_gathers(0)
    issue_idx(1, 1)

    @pl.loop(0, n_chunks, step=2)
    def _(qq):
      for ph in (0, 1):
        q = qq + ph
        wait_gathers(ph)
        wait_idx(1 - ph)
        fire_gathers(1 - ph)
        scatters(ph)
        issue_idx(q + 2, ph)

    wait_gathers(0)
    wait_idx(1)

    plsc.subcore_barrier()
    pltpu.sync_copy(acc.at[tile_slice], out_hbm.at[cid, tile_slice])

  return agg(table, src_rows, dst_rows, zeros)


def _blockdiag(w):
  din, dout = w.shape
  w8 = jnp.zeros((_D, _D), jnp.float32).at[:din, :dout].set(w)
  return jnp.kron(jnp.eye(_PK, dtype=jnp.float32), w8)


def _brow(b):
  b8 = jnp.zeros((_D,), jnp.float32).at[:b.shape[0]].set(b)
  return jnp.tile(b8, _PK).reshape(1, _LANES)


def _dot(a, b):
  return jax.lax.dot_general(a, b, (((1,), (0,)), ((), ())),
                             precision=jax.lax.Precision.HIGHEST)


def _combine1(p, xa_p, wl_blk, wr_blk, b_row, sel_deg):
  r = xa_p.shape[0]

  def body(pr, xar, wlr, wrr, br, selr, h_ref, dinv_ref):
    s = pr[0] + pr[1]
    dinv = 1.0 / jnp.maximum(_dot(s, selr[...]), 1.0)
    z = _dot(s * dinv, wlr[...]) + br[...] + _dot(xar[...], wrr[...])
    h_ref[...] = jnp.maximum(z, 0.0)
    dinv_ref[...] = dinv

  return pl.pallas_call(
      body,
      out_shape=[jax.ShapeDtypeStruct((r, _LANES), jnp.float32),
                 jax.ShapeDtypeStruct((r, _LANES), jnp.float32)],
  )(p, xa_p, wl_blk, wr_blk, b_row, sel_deg)


def _combine_mid(p, dinv_p, h_prev_p, wl_blk, wr_blk, b_row):
  r = h_prev_p.shape[0]

  def body(pr, dvr, hr, wlr, wrr, br, h_ref):
    mean = (pr[0] + pr[1]) * dvr[...]
    z = _dot(mean, wlr[...]) + br[...] + _dot(hr[...], wrr[...])
    h_ref[...] = jnp.maximum(z, 0.0)

  return pl.pallas_call(
      body,
      out_shape=jax.ShapeDtypeStruct((r, _LANES), jnp.float32),
  )(p, dinv_p, h_prev_p, wl_blk, wr_blk, b_row)


def _combine_last(p, dinv_p, h_prev_p, wl_blk, wr_blk, b_row, wc_blk, bc_row):
  r = h_prev_p.shape[0]

  def body(pr, dvr, hr, wlr, wrr, br, wcr, bcr, out_ref, h_ref):
    mean = (pr[0] + pr[1]) * dvr[...]
    z = _dot(mean, wlr[...]) + br[...] + _dot(hr[...], wrr[...])
    h = jnp.maximum(z, 0.0)
    h_ref[...] = h
    out_ref[...] = _dot(h, wcr[...]) + bcr[...]

  return pl.pallas_call(
      body,
      out_shape=[jax.ShapeDtypeStruct((r, _LANES), jnp.float32),
                 jax.ShapeDtypeStruct((r, _LANES), jnp.float32)],
  )(p, dinv_p, h_prev_p, wl_blk, wr_blk, b_row, wc_blk, bc_row)


def kernel(x, edge_index, w1l, b1, w1r, w2l, b2, w2r, w3l, b3, w3r, wc, bc):
  n = x.shape[0]
  e = edge_index.shape[1]

  group = _NC * _NS * _KB * _LANES * 2
  e_pad = ((e + group - 1) // group) * group
  n_acc = ((n + _LANES) // _LANES) * _LANES
  r = n_acc // _PK

  src = edge_index[0]
  dst = edge_index[1]
  npad = e_pad - e
  if npad:
    src = jnp.concatenate([src, jnp.zeros((npad,), jnp.int32)])
    pad_dst = n + (jnp.arange(npad, dtype=jnp.int32) % (n_acc - n))
    dst = jnp.concatenate([dst, pad_dst])
  src_rows = src.reshape(-1, _LANES)
  dst_rows = dst.reshape(-1, _LANES)

  xa = jnp.concatenate(
      [x, jnp.ones((n, 1), jnp.float32), jnp.zeros((n, _D - 2), jnp.float32)],
      axis=1)
  xa = jnp.concatenate([xa, jnp.zeros((n_acc - n, _D), jnp.float32)], axis=0)
  z8 = jnp.zeros((n_acc, _D), jnp.float32)

  sel8 = jnp.zeros((_D, _D), jnp.float32).at[1, :].set(1.0)
  sel_deg = jnp.kron(jnp.eye(_PK, dtype=jnp.float32), sel8)
  wl1, wr1, wl2, wr2, wl3, wr3, wcb = (
      _blockdiag(w) for w in (w1l, w1r, w2l, w2r, w3l, w3r, wc))
  br1, br2, br3, bcr = (_brow(b) for b in (b1, b2, b3, bc))

  xp = xa.reshape(r, _LANES)
  p = _sc_aggregate(xa, src_rows, dst_rows, z8, n_acc)
  h1p, dinvp = _combine1(p.reshape(_NC, r, _LANES), xp, wl1, wr1, br1, sel_deg)
  p = _sc_aggregate(h1p.reshape(n_acc, _D), src_rows, dst_rows, z8, n_acc)
  h2p = _combine_mid(p.reshape(_NC, r, _LANES), dinvp, h1p, wl2, wr2, br2)
  p = _sc_aggregate(h2p.reshape(n_acc, _D), src_rows, dst_rows, z8, n_acc)
  outp, h3p = _combine_last(p.reshape(_NC, r, _LANES), dinvp, h2p,
                            wl3, wr3, br3, wcb, bcr)
  out = outp.reshape(n_acc, _D)[:n, :2]
  h3 = h3p.reshape(n_acc, _D)[:n, :2]
  return (out, h3)
